# baseline (speedup 1.0000x reference)
import jax
import jax.numpy as jnp
from jax.experimental import pallas as pl

_S = 30.0
_M = 0.5
_BLOCK_R = 40


def _arcface_block(lab_ref, x_ref, o_ref):
    keep = lab_ref[...] >= 0
    o_ref[...] = jnp.where(keep, x_ref[...] * (1.0 - _S) + (_S * _M), 0.0)


def kernel(projected, labels, W):
    del W
    batch, out_f = projected.shape
    xt = projected.T
    lab = labels.reshape(1, batch)
    grid = (out_f // _BLOCK_R,)
    out_t = pl.pallas_call(
        _arcface_block,
        grid=grid,
        in_specs=[
            pl.BlockSpec((1, batch), lambda i: (0, 0)),
            pl.BlockSpec((_BLOCK_R, batch), lambda i: (i, 0)),
        ],
        out_specs=pl.BlockSpec((_BLOCK_R, batch), lambda i: (i, 0)),
        out_shape=jax.ShapeDtypeStruct((out_f, batch), projected.dtype),
    )(lab, xt)
    return out_t.T


# speedup vs baseline: 1.1298x; 1.1298x over previous
import jax
import jax.numpy as jnp
from jax.experimental import pallas as pl

_S = 30.0
_M = 0.5
_BLOCK_R = 200


def _arcface_block(lab_ref, x_ref, o_ref):
    keep = lab_ref[...] >= 0
    o_ref[...] = jnp.where(keep, x_ref[...] * (1.0 - _S) + (_S * _M), 0.0)


def kernel(projected, labels, W):
    del W
    batch, out_f = projected.shape
    xt = projected.T
    lab = labels.reshape(1, batch)
    grid = (out_f // _BLOCK_R,)
    out_t = pl.pallas_call(
        _arcface_block,
        grid=grid,
        in_specs=[
            pl.BlockSpec((1, batch), lambda i: (0, 0)),
            pl.BlockSpec((_BLOCK_R, batch), lambda i: (i, 0)),
        ],
        out_specs=pl.BlockSpec((_BLOCK_R, batch), lambda i: (i, 0)),
        out_shape=jax.ShapeDtypeStruct((out_f, batch), projected.dtype),
    )(lab, xt)
    return out_t.T
